# baseline (speedup 1.0000x reference)
import functools

import jax
import jax.numpy as jnp
from jax import lax
from jax.experimental import pallas as pl
from jax.experimental.pallas import tpu as pltpu
from jax.experimental.pallas import tpu_sc as plsc

N = 10000
D = 128
E = 320000

NCORES = 2
NSUB = 16
NWORK = NCORES * NSUB
EPW = E // NWORK
W = 40
NWIN = EPW // W
STRIPE = 624
TAIL = N - NSUB * STRIPE
NCPAD = 10240
CSTRIPE = NCPAD // NSUB
NB = 6


def _sc_agg_body(h_hbm, src_hbm, dst_hbm, z2_hbm, acc_out,
                 rows0, rows1, rows2, rows3, rows4, rows5,
                 swin0, swin1, swin2, swin3, swin4, swin5,
                 dwin0, dwin1, dwin2, dwin3, dwin4, dwin5, acc_s,
                 gsem0, gsem1, gsem2, gsem3, gsem4, gsem5,
                 ssem0, ssem1, ssem2, ssem3, ssem4, ssem5,
                 isem0, isem1, isem2, isem3, isem4, isem5):
    c = lax.axis_index("c")
    s = lax.axis_index("s")
    wid = s * NCORES + c
    rows = (rows0, rows1, rows2, rows3, rows4, rows5)
    swin = (swin0, swin1, swin2, swin3, swin4, swin5)
    dwin = (dwin0, dwin1, dwin2, dwin3, dwin4, dwin5)
    gsem = (gsem0, gsem1, gsem2, gsem3, gsem4, gsem5)
    ssem = (ssem0, ssem1, ssem2, ssem3, ssem4, ssem5)
    isem = (isem0, isem1, isem2, isem3, isem4, isem5)

    r0 = pl.multiple_of(s * STRIPE, 8)
    pltpu.sync_copy(z2_hbm.at[pl.ds(r0, STRIPE)], acc_s.at[pl.ds(r0, STRIPE)])

    @pl.when(s == 0)
    def _zero_tail():
        pltpu.sync_copy(z2_hbm.at[pl.ds(NSUB * STRIPE, TAIL)],
                        acc_s.at[pl.ds(NSUB * STRIPE, TAIL)])

    plsc.subcore_barrier()
    e0 = pl.multiple_of(wid * EPW, 16)

    def src_window(w):
        return src_hbm.at[pl.ds(e0 + pl.multiple_of(w * W, 8), W)]

    def dst_window(w):
        return dst_hbm.at[pl.ds(e0 + pl.multiple_of(w * W, 8), W)]

    def fetch_idx(w, q):
        pltpu.async_copy(src_window(w), swin[q], isem[q])
        pltpu.async_copy(dst_window(w), dwin[q], isem[q])

    def wait_idx(w, q):
        pltpu.make_async_copy(src_window(w), swin[q], isem[q]).wait()
        pltpu.make_async_copy(dst_window(w), dwin[q], isem[q]).wait()

    def start_gather(w, q):
        pltpu.async_copy(h_hbm.at[swin[q]], rows[q], gsem[q])

    def wait_scatter(w, q):
        pltpu.make_async_copy(rows[q], acc_s.at[dwin[q]], ssem[q]).wait()

    fetch_idx(0, 0)
    fetch_idx(1, 1)
    fetch_idx(2, 2)
    wait_idx(0, 0)
    start_gather(0, 0)
    wait_idx(1, 1)
    start_gather(1, 1)

    def stage(j, carry):
        for b in range(NB):
            w = NB * j + b

            @pl.when(w < NWIN)
            def _window():
                @pl.when(w >= 2)
                def _retire():
                    wait_scatter(w - 2, (b + NB - 2) % NB)

                @pl.when(w + 3 < NWIN)
                def _prefetch():
                    fetch_idx(w + 3, (b + 3) % NB)

                pltpu.make_async_copy(h_hbm.at[swin[b]], rows[b],
                                      gsem[b]).wait()

                pltpu.async_copy(rows[b], acc_s.at[dwin[b]], ssem[b],
                                 add=True)

                @pl.when(w + 2 < NWIN)
                def _next_gather():
                    wait_idx(w + 2, (b + 2) % NB)
                    start_gather(w + 2, (b + 2) % NB)
        return carry

    lax.fori_loop(0, (NWIN + NB - 1) // NB, stage, 0)
    wait_scatter(NWIN - 2, (NWIN - 2) % NB)
    wait_scatter(NWIN - 1, (NWIN - 1) % NB)
    plsc.subcore_barrier()

    pltpu.sync_copy(acc_s.at[pl.ds(r0, STRIPE)],
                    acc_out.at[c, pl.ds(r0, STRIPE)])

    @pl.when(s == 0)
    def _write_tail():
        pltpu.sync_copy(acc_s.at[pl.ds(NSUB * STRIPE, TAIL)],
                        acc_out.at[c, pl.ds(NSUB * STRIPE, TAIL)])


_sc_aggregate = pl.kernel(
    _sc_agg_body,
    out_type=jax.ShapeDtypeStruct((NCORES, N, D), jnp.float32),
    mesh=plsc.VectorSubcoreMesh(core_axis_name="c", subcore_axis_name="s"),
    scratch_types=(
        [pltpu.VMEM((W, D), jnp.float32)] * NB
        + [pltpu.VMEM((W,), jnp.int32)] * (2 * NB)
        + [pltpu.VMEM_SHARED((N, D), jnp.float32)]
        + [pltpu.SemaphoreType.DMA] * (3 * NB)
    ),
)

CW = 2000
CNWIN = EPW // CW


def _sc_cnt_body(dsta_hbm, dstb_hbm, z1_hbm, ones_hbm, cnta_out, cntb_out,
                 dwin0, dwin1, ones_v, cnta_s, cntb_s,
                 isem0, isem1, csem0, csem1):
    c = lax.axis_index("c")
    s = lax.axis_index("s")
    wid = s * NCORES + c
    dwin = (dwin0, dwin1)
    isem = (isem0, isem1)
    csem = (csem0, csem1)

    c0 = pl.multiple_of(s * CSTRIPE, 128)
    pltpu.sync_copy(z1_hbm.at[pl.ds(c0, CSTRIPE)], cnta_s.at[pl.ds(c0, CSTRIPE)])
    pltpu.sync_copy(z1_hbm.at[pl.ds(c0, CSTRIPE)], cntb_s.at[pl.ds(c0, CSTRIPE)])
    pltpu.sync_copy(ones_hbm, ones_v)
    plsc.subcore_barrier()

    e0 = pl.multiple_of(wid * EPW, 16)

    def win(ref, w):
        return ref.at[pl.ds(e0 + pl.multiple_of(w * CW, 16), CW)]

    for t, (dref, cnt_s) in enumerate(((dsta_hbm, cnta_s),
                                       (dstb_hbm, cntb_s))):
        pltpu.async_copy(win(dref, 0), dwin0, isem0)
        pltpu.async_copy(win(dref, 1), dwin1, isem1)

        def cstage(j, carry, dref=dref, cnt_s=cnt_s):
            for b in range(2):
                w = 2 * j + b

                @pl.when(w < CNWIN)
                def _cwindow():
                    pltpu.make_async_copy(win(dref, w), dwin[b],
                                          isem[b]).wait()
                    pltpu.async_copy(ones_v, cnt_s.at[dwin[b]], csem[b],
                                     add=True)
                    pltpu.make_async_copy(ones_v, cnt_s.at[dwin[b]],
                                          csem[b]).wait()

                    @pl.when(w + 2 < CNWIN)
                    def _next():
                        pltpu.async_copy(win(dref, w + 2), dwin[b],
                                         isem[b])
            return carry

        lax.fori_loop(0, (CNWIN + 1) // 2, cstage, 0)

    plsc.subcore_barrier()
    cbase = pl.multiple_of(c * NCPAD, 128)
    pltpu.sync_copy(cnta_s.at[pl.ds(c0, CSTRIPE)],
                    cnta_out.at[pl.ds(cbase + c0, CSTRIPE)])
    pltpu.sync_copy(cntb_s.at[pl.ds(c0, CSTRIPE)],
                    cntb_out.at[pl.ds(cbase + c0, CSTRIPE)])


_sc_counts = pl.kernel(
    _sc_cnt_body,
    out_type=[
        jax.ShapeDtypeStruct((NCORES * NCPAD,), jnp.float32),
        jax.ShapeDtypeStruct((NCORES * NCPAD,), jnp.float32),
    ],
    mesh=plsc.VectorSubcoreMesh(core_axis_name="c", subcore_axis_name="s"),
    scratch_types=(
        [pltpu.VMEM((CW,), jnp.int32)] * 2
        + [pltpu.VMEM((CW,), jnp.float32)]
        + [pltpu.VMEM_SHARED((NCPAD,), jnp.float32)] * 2
        + [pltpu.SemaphoreType.DMA] * 4
    ),
)


def _tc_body(h_ref, acc_ref, invb_ref, wlT_ref, wrT_ref, bl_ref, mask_ref,
             out_ref, *, apply_mask):
    mean = (acc_ref[0] + acc_ref[1]) * invb_ref[...]
    out = (jnp.dot(mean, wlT_ref[...], preferred_element_type=jnp.float32)
           + jnp.dot(h_ref[...], wrT_ref[...], preferred_element_type=jnp.float32)
           + bl_ref[...])
    if apply_mask:
        out = jnp.maximum(out * mask_ref[...], 0.0)
    out_ref[...] = out


RB = 1000


def _tc_layer(h, acc, invb, wlT, wrT, bl2d, mask, apply_mask):
    grid = (N // RB,)
    return pl.pallas_call(
        functools.partial(_tc_body, apply_mask=apply_mask),
        grid=grid,
        in_specs=[
            pl.BlockSpec((RB, D), lambda i: (i, 0)),
            pl.BlockSpec((NCORES, RB, D), lambda i: (0, i, 0)),
            pl.BlockSpec((RB, D), lambda i: (i, 0)),
            pl.BlockSpec((D, D), lambda i: (0, 0)),
            pl.BlockSpec((D, D), lambda i: (0, 0)),
            pl.BlockSpec((1, D), lambda i: (0, 0)),
            pl.BlockSpec((RB, D), lambda i: (i, 0)),
        ],
        out_specs=pl.BlockSpec((RB, D), lambda i: (i, 0)),
        out_shape=jax.ShapeDtypeStruct((N, D), jnp.float32),
    )(h, acc, invb, wlT, wrT, bl2d, mask)


def kernel(x, edge_index, edge_idx_1_1, Wl1, bl1, Wr1, Wl2, bl2, Wr2,
           Wl3, bl3, Wr3):
    f32 = jnp.float32
    z2 = jnp.zeros((N, D), f32)
    z1 = jnp.zeros((NCPAD,), f32)
    ones_w = jnp.ones((CW,), f32)

    src_a, dst_a = edge_index[0], edge_index[1]
    src_b, dst_b = edge_idx_1_1[0], edge_idx_1_1[1]

    cnt_a, cnt_b = _sc_counts(dst_a, dst_b, z1, ones_w)

    def inv_of(cnt):
        cnt = cnt.reshape(NCORES, NCPAD)[:, :N]
        inv = 1.0 / jnp.maximum(cnt[0] + cnt[1], 1.0)
        return jnp.broadcast_to(inv[:, None], (N, D))

    invb_a = inv_of(cnt_a)
    invb_b = inv_of(cnt_b)

    keep1 = jax.random.bernoulli(jax.random.key(1), 0.5, (N, D))
    keep2 = jax.random.bernoulli(jax.random.key(2), 0.5, (N, D))
    mask1 = keep1.astype(f32) * 2.0
    mask2 = keep2.astype(f32) * 2.0

    def layer(h, src, dst, invb, Wl, bl, Wr, mask, apply_mask):
        acc = _sc_aggregate(h, src, dst, z2)
        return _tc_layer(h, acc, invb, Wl.T, Wr.T, bl[None, :], mask,
                         apply_mask)

    h = layer(x, src_a, dst_a, invb_a, Wl1, bl1, Wr1, mask1, True)
    h = layer(h, src_b, dst_b, invb_b, Wl2, bl2, Wr2, mask2, True)
    h = layer(h, src_a, dst_a, invb_a, Wl3, bl3, Wr3, mask1, False)
    return h


# speedup vs baseline: 1.2927x; 1.2927x over previous
import functools

import jax
import jax.numpy as jnp
from jax import lax
from jax.experimental import pallas as pl
from jax.experimental.pallas import tpu as pltpu
from jax.experimental.pallas import tpu_sc as plsc

N = 10000
D = 128
E = 320000

NCORES = 2
NSUB = 16
NWORK = NCORES * NSUB
EPW = E // NWORK
W = 80
NWIN = EPW // W
STRIPE = 624
TAIL = N - NSUB * STRIPE
NCPAD = 10240
CSTRIPE = NCPAD // NSUB
NB = 4


def _sc_agg_body(h_hbm, src_hbm, dst_hbm, z2_hbm, acc_out,
                 rows0, rows1, rows2, rows3,
                 swin0, swin1, swin2, swin3,
                 dwin0, dwin1, dwin2, dwin3, acc_s,
                 gsem0, gsem1, gsem2, gsem3,
                 ssem0, ssem1, ssem2, ssem3,
                 isem0, isem1, isem2, isem3):
    c = lax.axis_index("c")
    s = lax.axis_index("s")
    wid = s * NCORES + c
    rows = (rows0, rows1, rows2, rows3)
    swin = (swin0, swin1, swin2, swin3)
    dwin = (dwin0, dwin1, dwin2, dwin3)
    gsem = (gsem0, gsem1, gsem2, gsem3)
    ssem = (ssem0, ssem1, ssem2, ssem3)
    isem = (isem0, isem1, isem2, isem3)

    r0 = pl.multiple_of(s * STRIPE, 8)
    pltpu.sync_copy(z2_hbm.at[pl.ds(r0, STRIPE)], acc_s.at[pl.ds(r0, STRIPE)])

    @pl.when(s == 0)
    def _zero_tail():
        pltpu.sync_copy(z2_hbm.at[pl.ds(NSUB * STRIPE, TAIL)],
                        acc_s.at[pl.ds(NSUB * STRIPE, TAIL)])

    plsc.subcore_barrier()
    e0 = pl.multiple_of(wid * EPW, 16)

    def src_window(w):
        return src_hbm.at[pl.ds(e0 + pl.multiple_of(w * W, 8), W)]

    def dst_window(w):
        return dst_hbm.at[pl.ds(e0 + pl.multiple_of(w * W, 8), W)]

    def fetch_idx(w, q):
        pltpu.async_copy(src_window(w), swin[q], isem[q])
        pltpu.async_copy(dst_window(w), dwin[q], isem[q])

    def wait_idx(w, q):
        pltpu.make_async_copy(src_window(w), swin[q], isem[q]).wait()
        pltpu.make_async_copy(dst_window(w), dwin[q], isem[q]).wait()

    def start_gather(w, q):
        pltpu.async_copy(h_hbm.at[swin[q]], rows[q], gsem[q])

    def wait_scatter(w, q):
        pltpu.make_async_copy(rows[q], acc_s.at[dwin[q]], ssem[q]).wait()

    fetch_idx(0, 0)
    fetch_idx(1, 1)
    wait_idx(0, 0)
    start_gather(0, 0)
    wait_idx(1, 1)
    start_gather(1, 1)

    def stage(j, carry):
        for b in range(NB):
            w = NB * j + b

            @pl.when(w < NWIN)
            def _window():
                @pl.when(w >= 2)
                def _retire():
                    wait_scatter(w - 2, (b + NB - 2) % NB)

                @pl.when(w + 2 < NWIN)
                def _prefetch():
                    fetch_idx(w + 2, (b + 2) % NB)

                pltpu.make_async_copy(h_hbm.at[swin[b]], rows[b],
                                      gsem[b]).wait()

                pltpu.async_copy(rows[b], acc_s.at[dwin[b]], ssem[b],
                                 add=True)

                @pl.when(w + 2 < NWIN)
                def _next_gather():
                    wait_idx(w + 2, (b + 2) % NB)
                    start_gather(w + 2, (b + 2) % NB)
        return carry

    lax.fori_loop(0, (NWIN + NB - 1) // NB, stage, 0)
    wait_scatter(NWIN - 2, (NWIN - 2) % NB)
    wait_scatter(NWIN - 1, (NWIN - 1) % NB)
    plsc.subcore_barrier()

    pltpu.sync_copy(acc_s.at[pl.ds(r0, STRIPE)],
                    acc_out.at[c, pl.ds(r0, STRIPE)])

    @pl.when(s == 0)
    def _write_tail():
        pltpu.sync_copy(acc_s.at[pl.ds(NSUB * STRIPE, TAIL)],
                        acc_out.at[c, pl.ds(NSUB * STRIPE, TAIL)])


_sc_aggregate = pl.kernel(
    _sc_agg_body,
    out_type=jax.ShapeDtypeStruct((NCORES, N, D), jnp.float32),
    mesh=plsc.VectorSubcoreMesh(core_axis_name="c", subcore_axis_name="s"),
    scratch_types=(
        [pltpu.VMEM((W, D), jnp.float32)] * NB
        + [pltpu.VMEM((W,), jnp.int32)] * (2 * NB)
        + [pltpu.VMEM_SHARED((N, D), jnp.float32)]
        + [pltpu.SemaphoreType.DMA] * (3 * NB)
    ),
)

CW = 2000
CNWIN = EPW // CW


def _sc_cnt_body(dsta_hbm, dstb_hbm, z1_hbm, ones_hbm, cnta_out, cntb_out,
                 dwin0, dwin1, ones_v, cnta_s, cntb_s,
                 isem0, isem1, csem0, csem1):
    c = lax.axis_index("c")
    s = lax.axis_index("s")
    wid = s * NCORES + c
    dwin = (dwin0, dwin1)
    isem = (isem0, isem1)
    csem = (csem0, csem1)

    c0 = pl.multiple_of(s * CSTRIPE, 128)
    pltpu.sync_copy(z1_hbm.at[pl.ds(c0, CSTRIPE)], cnta_s.at[pl.ds(c0, CSTRIPE)])
    pltpu.sync_copy(z1_hbm.at[pl.ds(c0, CSTRIPE)], cntb_s.at[pl.ds(c0, CSTRIPE)])
    pltpu.sync_copy(ones_hbm, ones_v)
    plsc.subcore_barrier()

    e0 = pl.multiple_of(wid * EPW, 16)

    def win(ref, w):
        return ref.at[pl.ds(e0 + pl.multiple_of(w * CW, 16), CW)]

    for t, (dref, cnt_s) in enumerate(((dsta_hbm, cnta_s),
                                       (dstb_hbm, cntb_s))):
        pltpu.async_copy(win(dref, 0), dwin0, isem0)
        pltpu.async_copy(win(dref, 1), dwin1, isem1)

        def cstage(j, carry, dref=dref, cnt_s=cnt_s):
            for b in range(2):
                w = 2 * j + b

                @pl.when(w < CNWIN)
                def _cwindow():
                    pltpu.make_async_copy(win(dref, w), dwin[b],
                                          isem[b]).wait()
                    pltpu.async_copy(ones_v, cnt_s.at[dwin[b]], csem[b],
                                     add=True)
                    pltpu.make_async_copy(ones_v, cnt_s.at[dwin[b]],
                                          csem[b]).wait()

                    @pl.when(w + 2 < CNWIN)
                    def _next():
                        pltpu.async_copy(win(dref, w + 2), dwin[b],
                                         isem[b])
            return carry

        lax.fori_loop(0, (CNWIN + 1) // 2, cstage, 0)

    plsc.subcore_barrier()
    cbase = pl.multiple_of(c * NCPAD, 128)
    pltpu.sync_copy(cnta_s.at[pl.ds(c0, CSTRIPE)],
                    cnta_out.at[pl.ds(cbase + c0, CSTRIPE)])
    pltpu.sync_copy(cntb_s.at[pl.ds(c0, CSTRIPE)],
                    cntb_out.at[pl.ds(cbase + c0, CSTRIPE)])


_sc_counts = pl.kernel(
    _sc_cnt_body,
    out_type=[
        jax.ShapeDtypeStruct((NCORES * NCPAD,), jnp.float32),
        jax.ShapeDtypeStruct((NCORES * NCPAD,), jnp.float32),
    ],
    mesh=plsc.VectorSubcoreMesh(core_axis_name="c", subcore_axis_name="s"),
    scratch_types=(
        [pltpu.VMEM((CW,), jnp.int32)] * 2
        + [pltpu.VMEM((CW,), jnp.float32)]
        + [pltpu.VMEM_SHARED((NCPAD,), jnp.float32)] * 2
        + [pltpu.SemaphoreType.DMA] * 4
    ),
)


def _tc_body(h_ref, acc_ref, invb_ref, wlT_ref, wrT_ref, bl_ref, mask_ref,
             out_ref, *, apply_mask):
    mean = (acc_ref[0] + acc_ref[1]) * invb_ref[...]
    out = (jnp.dot(mean, wlT_ref[...], preferred_element_type=jnp.float32)
           + jnp.dot(h_ref[...], wrT_ref[...], preferred_element_type=jnp.float32)
           + bl_ref[...])
    if apply_mask:
        out = jnp.maximum(out * mask_ref[...], 0.0)
    out_ref[...] = out


RB = 1000


def _tc_layer(h, acc, invb, wlT, wrT, bl2d, mask, apply_mask):
    grid = (N // RB,)
    return pl.pallas_call(
        functools.partial(_tc_body, apply_mask=apply_mask),
        grid=grid,
        in_specs=[
            pl.BlockSpec((RB, D), lambda i: (i, 0)),
            pl.BlockSpec((NCORES, RB, D), lambda i: (0, i, 0)),
            pl.BlockSpec((RB, D), lambda i: (i, 0)),
            pl.BlockSpec((D, D), lambda i: (0, 0)),
            pl.BlockSpec((D, D), lambda i: (0, 0)),
            pl.BlockSpec((1, D), lambda i: (0, 0)),
            pl.BlockSpec((RB, D), lambda i: (i, 0)),
        ],
        out_specs=pl.BlockSpec((RB, D), lambda i: (i, 0)),
        out_shape=jax.ShapeDtypeStruct((N, D), jnp.float32),
    )(h, acc, invb, wlT, wrT, bl2d, mask)


def kernel(x, edge_index, edge_idx_1_1, Wl1, bl1, Wr1, Wl2, bl2, Wr2,
           Wl3, bl3, Wr3):
    f32 = jnp.float32
    z2 = jnp.zeros((N, D), f32)
    z1 = jnp.zeros((NCPAD,), f32)
    ones_w = jnp.ones((CW,), f32)

    src_a, dst_a = edge_index[0], edge_index[1]
    src_b, dst_b = edge_idx_1_1[0], edge_idx_1_1[1]

    cnt_a, cnt_b = _sc_counts(dst_a, dst_b, z1, ones_w)

    def inv_of(cnt):
        cnt = cnt.reshape(NCORES, NCPAD)[:, :N]
        inv = 1.0 / jnp.maximum(cnt[0] + cnt[1], 1.0)
        return jnp.broadcast_to(inv[:, None], (N, D))

    invb_a = inv_of(cnt_a)
    invb_b = inv_of(cnt_b)

    keep1 = jax.random.bernoulli(jax.random.key(1), 0.5, (N, D))
    keep2 = jax.random.bernoulli(jax.random.key(2), 0.5, (N, D))
    mask1 = keep1.astype(f32) * 2.0
    mask2 = keep2.astype(f32) * 2.0

    def layer(h, src, dst, invb, Wl, bl, Wr, mask, apply_mask):
        acc = _sc_aggregate(h, src, dst, z2)
        return _tc_layer(h, acc, invb, Wl.T, Wr.T, bl[None, :], mask,
                         apply_mask)

    h = layer(x, src_a, dst_a, invb_a, Wl1, bl1, Wr1, mask1, True)
    h = layer(h, src_b, dst_b, invb_b, Wl2, bl2, Wr2, mask2, True)
    h = layer(h, src_a, dst_a, invb_a, Wl3, bl3, Wr3, mask1, False)
    return h


# speedup vs baseline: 1.3510x; 1.0450x over previous
import functools

import jax
import jax.numpy as jnp
from jax import lax
from jax.experimental import pallas as pl
from jax.experimental.pallas import tpu as pltpu
from jax.experimental.pallas import tpu_sc as plsc

N = 10000
D = 128
E = 320000

NCORES = 2
NSUB = 16
NWORK = NCORES * NSUB
EPW = E // NWORK
W = 80
NWIN = EPW // W
STRIPE = 624
TAIL = N - NSUB * STRIPE
NCPAD = 10240
CSTRIPE = NCPAD // NSUB
NB = 4


def _sc_agg_body(h_hbm, src_hbm, dst_hbm, z2_hbm, acc_out,
                 rows0, rows1, rows2, rows3,
                 swin0, swin1, swin2, swin3,
                 dwin0, dwin1, dwin2, dwin3, acc_s,
                 gsem0, gsem1, gsem2, gsem3,
                 ssem0, ssem1, ssem2, ssem3,
                 isem0, isem1, isem2, isem3):
    c = lax.axis_index("c")
    s = lax.axis_index("s")
    wid = s * NCORES + c
    rows = (rows0, rows1, rows2, rows3)
    swin = (swin0, swin1, swin2, swin3)
    dwin = (dwin0, dwin1, dwin2, dwin3)
    gsem = (gsem0, gsem1, gsem2, gsem3)
    ssem = (ssem0, ssem1, ssem2, ssem3)
    isem = (isem0, isem1, isem2, isem3)

    r0 = pl.multiple_of(s * STRIPE, 8)
    pltpu.sync_copy(z2_hbm.at[pl.ds(r0, STRIPE)], acc_s.at[pl.ds(r0, STRIPE)])

    @pl.when(s == 0)
    def _zero_tail():
        pltpu.sync_copy(z2_hbm.at[pl.ds(NSUB * STRIPE, TAIL)],
                        acc_s.at[pl.ds(NSUB * STRIPE, TAIL)])

    plsc.subcore_barrier()
    e0 = pl.multiple_of(wid * EPW, 16)

    def src_window(w):
        return src_hbm.at[pl.ds(e0 + pl.multiple_of(w * W, 8), W)]

    def dst_window(w):
        return dst_hbm.at[pl.ds(e0 + pl.multiple_of(w * W, 8), W)]

    def fetch_idx(w, q):
        pltpu.async_copy(src_window(w), swin[q], isem[q])
        pltpu.async_copy(dst_window(w), dwin[q], isem[q])

    def wait_idx(w, q):
        pltpu.make_async_copy(src_window(w), swin[q], isem[q]).wait()
        pltpu.make_async_copy(dst_window(w), dwin[q], isem[q]).wait()

    def start_gather(w, q):
        pltpu.async_copy(h_hbm.at[swin[q]], rows[q], gsem[q])

    def wait_scatter(w, q):
        pltpu.make_async_copy(rows[q], acc_s.at[dwin[q]], ssem[q]).wait()

    fetch_idx(0, 0)
    fetch_idx(1, 1)
    wait_idx(0, 0)
    start_gather(0, 0)
    wait_idx(1, 1)
    start_gather(1, 1)

    def stage(j, carry):
        for b in range(NB):
            w = NB * j + b

            @pl.when(w < NWIN)
            def _window():
                @pl.when(w >= 2)
                def _retire():
                    wait_scatter(w - 2, (b + NB - 2) % NB)

                @pl.when(w + 2 < NWIN)
                def _prefetch():
                    fetch_idx(w + 2, (b + 2) % NB)

                pltpu.make_async_copy(h_hbm.at[swin[b]], rows[b],
                                      gsem[b]).wait()

                pltpu.async_copy(rows[b], acc_s.at[dwin[b]], ssem[b],
                                 add=True)

                @pl.when(w + 2 < NWIN)
                def _next_gather():
                    wait_idx(w + 2, (b + 2) % NB)
                    start_gather(w + 2, (b + 2) % NB)
        return carry

    lax.fori_loop(0, (NWIN + NB - 1) // NB, stage, 0)
    wait_scatter(NWIN - 2, (NWIN - 2) % NB)
    wait_scatter(NWIN - 1, (NWIN - 1) % NB)
    plsc.subcore_barrier()

    pltpu.sync_copy(acc_s.at[pl.ds(r0, STRIPE)],
                    acc_out.at[c, pl.ds(r0, STRIPE)])

    @pl.when(s == 0)
    def _write_tail():
        pltpu.sync_copy(acc_s.at[pl.ds(NSUB * STRIPE, TAIL)],
                        acc_out.at[c, pl.ds(NSUB * STRIPE, TAIL)])


_sc_aggregate = pl.kernel(
    _sc_agg_body,
    out_type=jax.ShapeDtypeStruct((NCORES, N, D), jnp.float32),
    mesh=plsc.VectorSubcoreMesh(core_axis_name="c", subcore_axis_name="s"),
    scratch_types=(
        [pltpu.VMEM((W, D), jnp.float32)] * NB
        + [pltpu.VMEM((W,), jnp.int32)] * (2 * NB)
        + [pltpu.VMEM_SHARED((N, D), jnp.float32)]
        + [pltpu.SemaphoreType.DMA] * (3 * NB)
    ),
)

CW = 2000
CNWIN = EPW // CW


def _sc_cnt_body(dsta_hbm, dstb_hbm, z1_hbm, ones_hbm, cnta_out, cntb_out,
                 dwin0, dwin1, ones_v, cnta_s, cntb_s,
                 isem0, isem1, csem0, csem1):
    c = lax.axis_index("c")
    s = lax.axis_index("s")
    wid = s * NCORES + c
    dwin = (dwin0, dwin1)
    isem = (isem0, isem1)
    csem = (csem0, csem1)

    c0 = pl.multiple_of(s * CSTRIPE, 128)
    pltpu.sync_copy(z1_hbm.at[pl.ds(c0, CSTRIPE)], cnta_s.at[pl.ds(c0, CSTRIPE)])
    pltpu.sync_copy(z1_hbm.at[pl.ds(c0, CSTRIPE)], cntb_s.at[pl.ds(c0, CSTRIPE)])
    pltpu.sync_copy(ones_hbm, ones_v)
    plsc.subcore_barrier()

    e0 = pl.multiple_of(wid * EPW, 16)

    def win(ref, w):
        return ref.at[pl.ds(e0 + pl.multiple_of(w * CW, 16), CW)]

    for t, (dref, cnt_s) in enumerate(((dsta_hbm, cnta_s),
                                       (dstb_hbm, cntb_s))):
        pltpu.async_copy(win(dref, 0), dwin0, isem0)
        pltpu.async_copy(win(dref, 1), dwin1, isem1)

        def cstage(j, carry, dref=dref, cnt_s=cnt_s):
            for b in range(2):
                w = 2 * j + b

                @pl.when(w < CNWIN)
                def _cwindow():
                    pltpu.make_async_copy(win(dref, w), dwin[b],
                                          isem[b]).wait()
                    pltpu.async_copy(ones_v, cnt_s.at[dwin[b]], csem[b],
                                     add=True)
                    pltpu.make_async_copy(ones_v, cnt_s.at[dwin[b]],
                                          csem[b]).wait()

                    @pl.when(w + 2 < CNWIN)
                    def _next():
                        pltpu.async_copy(win(dref, w + 2), dwin[b],
                                         isem[b])
            return carry

        lax.fori_loop(0, (CNWIN + 1) // 2, cstage, 0)

    plsc.subcore_barrier()
    cbase = pl.multiple_of(c * NCPAD, 128)
    pltpu.sync_copy(cnta_s.at[pl.ds(c0, CSTRIPE)],
                    cnta_out.at[pl.ds(cbase + c0, CSTRIPE)])
    pltpu.sync_copy(cntb_s.at[pl.ds(c0, CSTRIPE)],
                    cntb_out.at[pl.ds(cbase + c0, CSTRIPE)])


_sc_counts = pl.kernel(
    _sc_cnt_body,
    out_type=[
        jax.ShapeDtypeStruct((NCORES * NCPAD,), jnp.float32),
        jax.ShapeDtypeStruct((NCORES * NCPAD,), jnp.float32),
    ],
    mesh=plsc.VectorSubcoreMesh(core_axis_name="c", subcore_axis_name="s"),
    scratch_types=(
        [pltpu.VMEM((CW,), jnp.int32)] * 2
        + [pltpu.VMEM((CW,), jnp.float32)]
        + [pltpu.VMEM_SHARED((NCPAD,), jnp.float32)] * 2
        + [pltpu.SemaphoreType.DMA] * 4
    ),
)


def _tc_body(h_ref, acc_ref, invb_ref, wlT_ref, wrT_ref, bl_ref, mask_ref,
             out_ref, *, apply_mask):
    mean = (acc_ref[0] + acc_ref[1]) * invb_ref[...]
    out = (jnp.dot(mean, wlT_ref[...], preferred_element_type=jnp.float32)
           + jnp.dot(h_ref[...], wrT_ref[...], preferred_element_type=jnp.float32)
           + bl_ref[...])
    if apply_mask:
        out = jnp.maximum(out * mask_ref[...], 0.0)
    out_ref[...] = out


RB = 2000


def _tc_layer(h, acc, invb, wlT, wrT, bl2d, mask, apply_mask):
    grid = (N // RB,)
    return pl.pallas_call(
        functools.partial(_tc_body, apply_mask=apply_mask),
        grid=grid,
        in_specs=[
            pl.BlockSpec((RB, D), lambda i: (i, 0)),
            pl.BlockSpec((NCORES, RB, D), lambda i: (0, i, 0)),
            pl.BlockSpec((RB, 1), lambda i: (i, 0)),
            pl.BlockSpec((D, D), lambda i: (0, 0)),
            pl.BlockSpec((D, D), lambda i: (0, 0)),
            pl.BlockSpec((1, D), lambda i: (0, 0)),
            pl.BlockSpec((RB, D), lambda i: (i, 0)),
        ],
        out_specs=pl.BlockSpec((RB, D), lambda i: (i, 0)),
        out_shape=jax.ShapeDtypeStruct((N, D), jnp.float32),
    )(h, acc, invb, wlT, wrT, bl2d, mask)


def kernel(x, edge_index, edge_idx_1_1, Wl1, bl1, Wr1, Wl2, bl2, Wr2,
           Wl3, bl3, Wr3):
    f32 = jnp.float32
    z2 = jnp.zeros((N, D), f32)
    z1 = jnp.zeros((NCPAD,), f32)
    ones_w = jnp.ones((CW,), f32)

    src_a, dst_a = edge_index[0], edge_index[1]
    src_b, dst_b = edge_idx_1_1[0], edge_idx_1_1[1]

    cnt_a, cnt_b = _sc_counts(dst_a, dst_b, z1, ones_w)

    def inv_of(cnt):
        cnt = cnt.reshape(NCORES, NCPAD)[:, :N]
        inv = 1.0 / jnp.maximum(cnt[0] + cnt[1], 1.0)
        return inv[:, None]

    invb_a = inv_of(cnt_a)
    invb_b = inv_of(cnt_b)

    keep1 = jax.random.bernoulli(jax.random.key(1), 0.5, (N, D))
    keep2 = jax.random.bernoulli(jax.random.key(2), 0.5, (N, D))
    mask1 = keep1.astype(f32) * 2.0
    mask2 = keep2.astype(f32) * 2.0

    def layer(h, src, dst, invb, Wl, bl, Wr, mask, apply_mask):
        acc = _sc_aggregate(h, src, dst, z2)
        return _tc_layer(h, acc, invb, Wl.T, Wr.T, bl[None, :], mask,
                         apply_mask)

    h = layer(x, src_a, dst_a, invb_a, Wl1, bl1, Wr1, mask1, True)
    h = layer(h, src_b, dst_b, invb_b, Wl2, bl2, Wr2, mask2, True)
    h = layer(h, src_a, dst_a, invb_a, Wl3, bl3, Wr3, mask1, False)
    return h


# speedup vs baseline: 1.3518x; 1.0006x over previous
import functools

import jax
import jax.numpy as jnp
from jax import lax
from jax.experimental import pallas as pl
from jax.experimental.pallas import tpu as pltpu
from jax.experimental.pallas import tpu_sc as plsc

N = 10000
D = 128
E = 320000

NCORES = 2
NSUB = 16
NWORK = NCORES * NSUB
EPW = E // NWORK
W = 80
NWIN = EPW // W
STRIPE = 624
TAIL = N - NSUB * STRIPE
NCPAD = 10240
CSTRIPE = NCPAD // NSUB
NB = 4


def _sc_agg_body(h_hbm, src_hbm, dst_hbm, z2_hbm, acc_out,
                 rows0, rows1, rows2, rows3,
                 swin0, swin1, swin2, swin3,
                 dwin0, dwin1, dwin2, dwin3, acc_s,
                 gsem0, gsem1, gsem2, gsem3,
                 ssem0, ssem1, ssem2, ssem3,
                 isem0, isem1, isem2, isem3):
    c = lax.axis_index("c")
    s = lax.axis_index("s")
    wid = s * NCORES + c
    rows = (rows0, rows1, rows2, rows3)
    swin = (swin0, swin1, swin2, swin3)
    dwin = (dwin0, dwin1, dwin2, dwin3)
    gsem = (gsem0, gsem1, gsem2, gsem3)
    ssem = (ssem0, ssem1, ssem2, ssem3)
    isem = (isem0, isem1, isem2, isem3)

    r0 = pl.multiple_of(s * STRIPE, 8)
    pltpu.sync_copy(z2_hbm.at[pl.ds(r0, STRIPE)], acc_s.at[pl.ds(r0, STRIPE)])

    @pl.when(s == 0)
    def _zero_tail():
        pltpu.sync_copy(z2_hbm.at[pl.ds(NSUB * STRIPE, TAIL)],
                        acc_s.at[pl.ds(NSUB * STRIPE, TAIL)])

    plsc.subcore_barrier()
    e0 = pl.multiple_of(wid * EPW, 16)

    def src_window(w):
        return src_hbm.at[pl.ds(e0 + pl.multiple_of(w * W, 8), W)]

    def dst_window(w):
        return dst_hbm.at[pl.ds(e0 + pl.multiple_of(w * W, 8), W)]

    def fetch_idx(w, q):
        pltpu.async_copy(src_window(w), swin[q], isem[q])
        pltpu.async_copy(dst_window(w), dwin[q], isem[q])

    def wait_idx(w, q):
        pltpu.make_async_copy(src_window(w), swin[q], isem[q]).wait()
        pltpu.make_async_copy(dst_window(w), dwin[q], isem[q]).wait()

    def start_gather(w, q):
        pltpu.async_copy(h_hbm.at[swin[q]], rows[q], gsem[q])

    def wait_scatter(w, q):
        pltpu.make_async_copy(rows[q], acc_s.at[dwin[q]], ssem[q]).wait()

    fetch_idx(0, 0)
    fetch_idx(1, 1)
    wait_idx(0, 0)
    start_gather(0, 0)
    wait_idx(1, 1)
    start_gather(1, 1)

    def stage(j, carry):
        for b in range(NB):
            w = NB * j + b

            @pl.when(w < NWIN)
            def _window():
                @pl.when(w >= 2)
                def _retire():
                    wait_scatter(w - 2, (b + NB - 2) % NB)

                @pl.when(w + 2 < NWIN)
                def _prefetch():
                    fetch_idx(w + 2, (b + 2) % NB)

                pltpu.make_async_copy(h_hbm.at[swin[b]], rows[b],
                                      gsem[b]).wait()

                pltpu.async_copy(rows[b], acc_s.at[dwin[b]], ssem[b],
                                 add=True)

                @pl.when(w + 2 < NWIN)
                def _next_gather():
                    wait_idx(w + 2, (b + 2) % NB)
                    start_gather(w + 2, (b + 2) % NB)
        return carry

    lax.fori_loop(0, (NWIN + NB - 1) // NB, stage, 0)
    wait_scatter(NWIN - 2, (NWIN - 2) % NB)
    wait_scatter(NWIN - 1, (NWIN - 1) % NB)
    plsc.subcore_barrier()

    pltpu.sync_copy(acc_s.at[pl.ds(r0, STRIPE)],
                    acc_out.at[c, pl.ds(r0, STRIPE)])

    @pl.when(s == 0)
    def _write_tail():
        pltpu.sync_copy(acc_s.at[pl.ds(NSUB * STRIPE, TAIL)],
                        acc_out.at[c, pl.ds(NSUB * STRIPE, TAIL)])


_sc_aggregate = pl.kernel(
    _sc_agg_body,
    out_type=jax.ShapeDtypeStruct((NCORES, N, D), jnp.float32),
    mesh=plsc.VectorSubcoreMesh(core_axis_name="c", subcore_axis_name="s"),
    scratch_types=(
        [pltpu.VMEM((W, D), jnp.float32)] * NB
        + [pltpu.VMEM((W,), jnp.int32)] * (2 * NB)
        + [pltpu.VMEM_SHARED((N, D), jnp.float32)]
        + [pltpu.SemaphoreType.DMA] * (3 * NB)
    ),
)

CW = 2000
CNWIN = EPW // CW


def _sc_cnt_body(dsta_hbm, dstb_hbm, z1_hbm, ones_hbm, cnta_out, cntb_out,
                 dwin0, dwin1, ones_v, cnta_s, cntb_s,
                 isem0, isem1, csem0, csem1):
    c = lax.axis_index("c")
    s = lax.axis_index("s")
    wid = s * NCORES + c
    dwin = (dwin0, dwin1)
    isem = (isem0, isem1)
    csem = (csem0, csem1)

    c0 = pl.multiple_of(s * CSTRIPE, 128)
    pltpu.sync_copy(z1_hbm.at[pl.ds(c0, CSTRIPE)], cnta_s.at[pl.ds(c0, CSTRIPE)])
    pltpu.sync_copy(z1_hbm.at[pl.ds(c0, CSTRIPE)], cntb_s.at[pl.ds(c0, CSTRIPE)])
    pltpu.sync_copy(ones_hbm, ones_v)
    plsc.subcore_barrier()

    e0 = pl.multiple_of(wid * EPW, 16)

    def win(ref, w):
        return ref.at[pl.ds(e0 + pl.multiple_of(w * CW, 16), CW)]

    for dref, cnt_s in ((dsta_hbm, cnta_s), (dstb_hbm, cntb_s)):
        pltpu.async_copy(win(dref, 0), dwin0, isem0)
        pltpu.async_copy(win(dref, 1), dwin1, isem1)

        def cstage(j, carry, dref=dref, cnt_s=cnt_s):
            for b in range(2):
                w = 2 * j + b

                @pl.when(w < CNWIN)
                def _cwindow():
                    pltpu.make_async_copy(win(dref, w), dwin[b],
                                          isem[b]).wait()
                    pltpu.async_copy(ones_v, cnt_s.at[dwin[b]], csem[b],
                                     add=True)
                    pltpu.make_async_copy(ones_v, cnt_s.at[dwin[b]],
                                          csem[b]).wait()

                    @pl.when(w + 2 < CNWIN)
                    def _next():
                        pltpu.async_copy(win(dref, w + 2), dwin[b],
                                         isem[b])
            return carry

        lax.fori_loop(0, (CNWIN + 1) // 2, cstage, 0)

    plsc.subcore_barrier()
    cbase = pl.multiple_of(c * NCPAD, 128)
    pltpu.sync_copy(cnta_s.at[pl.ds(c0, CSTRIPE)],
                    cnta_out.at[pl.ds(cbase + c0, CSTRIPE)])
    pltpu.sync_copy(cntb_s.at[pl.ds(c0, CSTRIPE)],
                    cntb_out.at[pl.ds(cbase + c0, CSTRIPE)])


_sc_counts = pl.kernel(
    _sc_cnt_body,
    out_type=[
        jax.ShapeDtypeStruct((NCORES * NCPAD,), jnp.float32),
        jax.ShapeDtypeStruct((NCORES * NCPAD,), jnp.float32),
    ],
    mesh=plsc.VectorSubcoreMesh(core_axis_name="c", subcore_axis_name="s"),
    scratch_types=(
        [pltpu.VMEM((CW,), jnp.int32)] * 2
        + [pltpu.VMEM((CW,), jnp.float32)]
        + [pltpu.VMEM_SHARED((NCPAD,), jnp.float32)] * 2
        + [pltpu.SemaphoreType.DMA] * 4
    ),
)


def _tc_body(h_ref, acc_ref, invb_ref, wlT_ref, wrT_ref, bl_ref, mask_ref,
             out_ref, *, apply_mask):
    mean = (acc_ref[0] + acc_ref[1]) * invb_ref[...]
    out = (jnp.dot(mean, wlT_ref[...], preferred_element_type=jnp.float32)
           + jnp.dot(h_ref[...], wrT_ref[...], preferred_element_type=jnp.float32)
           + bl_ref[...])
    if apply_mask:
        out = jnp.maximum(out * mask_ref[...], 0.0)
    out_ref[...] = out


RB = 2000


def _tc_layer(h, acc, invb, wlT, wrT, bl2d, mask, apply_mask):
    grid = (N // RB,)
    return pl.pallas_call(
        functools.partial(_tc_body, apply_mask=apply_mask),
        grid=grid,
        in_specs=[
            pl.BlockSpec((RB, D), lambda i: (i, 0)),
            pl.BlockSpec((NCORES, RB, D), lambda i: (0, i, 0)),
            pl.BlockSpec((RB, 1), lambda i: (i, 0)),
            pl.BlockSpec((D, D), lambda i: (0, 0)),
            pl.BlockSpec((D, D), lambda i: (0, 0)),
            pl.BlockSpec((1, D), lambda i: (0, 0)),
            pl.BlockSpec((RB, D), lambda i: (i, 0)),
        ],
        out_specs=pl.BlockSpec((RB, D), lambda i: (i, 0)),
        out_shape=jax.ShapeDtypeStruct((N, D), jnp.float32),
    )(h, acc, invb, wlT, wrT, bl2d, mask)


def kernel(x, edge_index, edge_idx_1_1, Wl1, bl1, Wr1, Wl2, bl2, Wr2,
           Wl3, bl3, Wr3):
    f32 = jnp.float32
    z2 = jnp.zeros((N, D), f32)
    z1 = jnp.zeros((NCPAD,), f32)
    ones_w = jnp.ones((CW,), f32)

    src_a, dst_a = edge_index[0], edge_index[1]
    src_b, dst_b = edge_idx_1_1[0], edge_idx_1_1[1]

    cnt_a, cnt_b = _sc_counts(dst_a, dst_b, z1, ones_w)

    def inv_of(cnt):
        cnt = cnt.reshape(NCORES, NCPAD)[:, :N]
        inv = 1.0 / jnp.maximum(cnt[0] + cnt[1], 1.0)
        return inv[:, None]

    invb_a = inv_of(cnt_a)
    invb_b = inv_of(cnt_b)

    keep1 = jax.random.bernoulli(jax.random.key(1), 0.5, (N, D))
    keep2 = jax.random.bernoulli(jax.random.key(2), 0.5, (N, D))
    mask1 = keep1.astype(f32) * 2.0
    mask2 = keep2.astype(f32) * 2.0

    def layer(h, src, dst, invb, Wl, bl, Wr, mask, apply_mask):
        acc = _sc_aggregate(h, src, dst, z2)
        return _tc_layer(h, acc, invb, Wl.T, Wr.T, bl[None, :], mask,
                         apply_mask)

    h = layer(x, src_a, dst_a, invb_a, Wl1, bl1, Wr1, mask1, True)
    h = layer(h, src_b, dst_b, invb_b, Wl2, bl2, Wr2, mask2, True)
    h = layer(h, src_a, dst_a, invb_a, Wl3, bl3, Wr3, mask1, False)
    return h
